# baseline (speedup 1.0000x reference)
import jax
import jax.numpy as jnp
from jax.experimental import pallas as pl
from jax.experimental.pallas import tpu as pltpu
from jax.experimental.pallas import tpu_sc as plsc

S = 2048
D = 576
H = 9
HD = D // H
LAT = D // 4
INTER = 1536
NR = 7
EPS = 1e-5

BA = 512
BQ = 512
BK = 512
BC = 512
FB = 512
NF = INTER // FB

_F32 = jnp.float32


def _rope(t, cos, sin):
    parts = []
    for h in range(H):
        th = t[:, h * HD:(h + 1) * HD]
        rot = jnp.concatenate([-th[:, HD // 2:], th[:, :HD // 2]], axis=1)
        parts.append(th * cos + rot * sin)
    return jnp.concatenate(parts, axis=1)


def _qkv_body(x_ref, ln1_ref, wqd_ref, wqu_ref, wkvd_ref, wku_ref, wvu_ref,
              q_ref, k_ref, v_ref):
    i = pl.program_id(0)
    xb = x_ref[...]
    h = xb * jax.lax.rsqrt(jnp.mean(xb * xb, axis=1, keepdims=True) + EPS)
    h = h * ln1_ref[...]
    q = jnp.dot(jnp.dot(h, wqd_ref[...], preferred_element_type=_F32),
                wqu_ref[...], preferred_element_type=_F32)
    kv = jnp.dot(h, wkvd_ref[...], preferred_element_type=_F32)
    k = jnp.dot(kv, wku_ref[...], preferred_element_type=_F32)
    v = jnp.dot(kv, wvu_ref[...], preferred_element_type=_F32)
    pos = (i * BA + jax.lax.broadcasted_iota(jnp.int32, (BA, HD), 0)).astype(_F32)
    lane = jax.lax.broadcasted_iota(jnp.int32, (BA, HD), 1)
    kk = jnp.where(lane < HD // 2, lane, lane - HD // 2).astype(_F32)
    inv = jnp.exp(kk * (-2.0 * jnp.log(10000.0) / HD))
    ang = pos * inv
    cos = jnp.cos(ang)
    sin = jnp.sin(ang)
    q_ref[...] = _rope(q, cos, sin).astype(jnp.bfloat16)
    k_ref[...] = _rope(k, cos, sin).astype(jnp.bfloat16)
    v_ref[...] = v.astype(jnp.bfloat16)


def _qkv(xf, ln1, wq_d, wq_u, wkv_d, wk_u, wv_u):
    out = jax.ShapeDtypeStruct((S, D), jnp.bfloat16)
    row_spec = pl.BlockSpec((BA, D), lambda i: (i, 0))
    return pl.pallas_call(
        _qkv_body,
        grid=(S // BA,),
        in_specs=[
            row_spec,
            pl.BlockSpec((1, D), lambda i: (0, 0)),
            pl.BlockSpec((D, LAT), lambda i: (0, 0)),
            pl.BlockSpec((LAT, D), lambda i: (0, 0)),
            pl.BlockSpec((D, LAT), lambda i: (0, 0)),
            pl.BlockSpec((LAT, D), lambda i: (0, 0)),
            pl.BlockSpec((LAT, D), lambda i: (0, 0)),
        ],
        out_specs=[row_spec, row_spec, row_spec],
        out_shape=[out, out, out],
    )(xf, ln1, wq_d, wq_u, wkv_d, wk_u, wv_u)


def _attn_body(q_ref, k_ref, v_ref, o_ref):
    i = pl.program_id(0)
    rows = jax.lax.broadcasted_iota(jnp.int32, (BQ, BK), 0)
    cols = jax.lax.broadcasted_iota(jnp.int32, (BQ, BK), 1)
    for h in range(H):
        hs = slice(h * HD, (h + 1) * HD)
        qh = q_ref[:, hs]

        def body(j, carry, qh=qh, hs=hs):
            m, l, acc = carry
            kh = k_ref[pl.ds(j * BK, BK), hs]
            lg = jax.lax.dot_general(qh, kh, (((1,), (1,)), ((), ())),
                                     preferred_element_type=_F32) * 0.125
            mask = (i * BQ + rows) >= (j * BK + cols)
            lg = jnp.where(mask, lg, -1e30)
            mj = jnp.max(lg, axis=1, keepdims=True)
            mn = jnp.maximum(m, mj)
            p = jnp.exp(lg - mn)
            scale = jnp.exp(m - mn)
            l2 = l * scale + jnp.sum(p, axis=1, keepdims=True)
            acc2 = acc * scale + jnp.dot(
                p.astype(jnp.bfloat16), v_ref[pl.ds(j * BK, BK), hs],
                preferred_element_type=_F32)
            return mn, l2, acc2

        m0 = jnp.full((BQ, 1), -1e30, _F32)
        l0 = jnp.zeros((BQ, 1), _F32)
        a0 = jnp.zeros((BQ, HD), _F32)
        m, l, acc = jax.lax.fori_loop(0, i + 1, body, (m0, l0, a0))
        o_ref[:, hs] = (acc / l).astype(jnp.bfloat16)


def _attention(q, k, v):
    row_spec = pl.BlockSpec((BQ, D), lambda i: (i, 0))
    kv_spec = pl.BlockSpec((S, D), lambda i: (0, 0))
    return pl.pallas_call(
        _attn_body,
        grid=(S // BQ,),
        in_specs=[row_spec, kv_spec, kv_spec],
        out_specs=row_spec,
        out_shape=jax.ShapeDtypeStruct((S, D), jnp.bfloat16),
    )(q, k, v)


def _post_body(x_ref, attn_ref, wo_ref, ln2_ref, sg_ref, su_ref, sd_ref,
               rw_ref, rb_ref, part_ref, h2_ref, w_ref, ind_ref):
    x2 = x_ref[...] + jnp.dot(attn_ref[...],
                              wo_ref[...].astype(jnp.bfloat16),
                              preferred_element_type=_F32)
    h2 = x2 * jax.lax.rsqrt(jnp.mean(x2 * x2, axis=1, keepdims=True) + EPS)
    h2 = h2 * ln2_ref[...]
    h2b = h2.astype(jnp.bfloat16)
    g = jnp.dot(h2b, sg_ref[...].astype(jnp.bfloat16),
                preferred_element_type=_F32)
    u = jnp.dot(h2b, su_ref[...].astype(jnp.bfloat16),
                preferred_element_type=_F32)
    a = g * jax.nn.sigmoid(g) * u
    shared = jnp.dot(a.astype(jnp.bfloat16), sd_ref[...].astype(jnp.bfloat16),
                     preferred_element_type=_F32)
    part_ref[...] = x2 + shared
    h2_ref[...] = jnp.concatenate([h2, jnp.zeros((BC, DP - D), _F32)], axis=1)
    logits = jnp.dot(h2, rw_ref[...], preferred_element_type=_F32) + rb_ref[...]
    p = jax.nn.sigmoid(logits)
    colid = jax.lax.broadcasted_iota(jnp.int32, (BC, 8), 1)
    p = jnp.where(colid < NR, p, -1.0)
    m1 = jnp.max(p, axis=1, keepdims=True)
    i1 = jnp.min(jnp.where(p == m1, colid, 127), axis=1, keepdims=True)
    pm = jnp.where(colid == i1, -1.0, p)
    m2 = jnp.max(pm, axis=1, keepdims=True)
    i2 = jnp.min(jnp.where(pm == m2, colid, 127), axis=1, keepdims=True)
    den = m1 + m2
    w_ref[...] = (jnp.where(colid == i1, m1, 0.0)
                  + jnp.where(colid == i2, m2, 0.0)) / den
    ind_ref[...] = (jnp.where(colid == i1, 1.0, 0.0)
                    + jnp.where(colid == i2, 1.0, 0.0))


def _post(xf, attn, wo, ln2, s_gate, s_up, s_down, rw, rb):
    row_spec = pl.BlockSpec((BC, D), lambda i: (i, 0))
    return pl.pallas_call(
        _post_body,
        grid=(S // BC,),
        in_specs=[
            row_spec,
            row_spec,
            pl.BlockSpec((D, D), lambda i: (0, 0)),
            pl.BlockSpec((1, D), lambda i: (0, 0)),
            pl.BlockSpec((D, INTER), lambda i: (0, 0)),
            pl.BlockSpec((D, INTER), lambda i: (0, 0)),
            pl.BlockSpec((INTER, D), lambda i: (0, 0)),
            pl.BlockSpec((D, 8), lambda i: (0, 0)),
            pl.BlockSpec((1, 8), lambda i: (0, 0)),
        ],
        out_specs=[row_spec, pl.BlockSpec((BC, DP), lambda i: (i, 0)),
                   pl.BlockSpec((BC, 8), lambda i: (i, 0)),
                   pl.BlockSpec((BC, 8), lambda i: (i, 0))],
        out_shape=[
            jax.ShapeDtypeStruct((S, D), _F32),
            jax.ShapeDtypeStruct((S, DP), _F32),
            jax.ShapeDtypeStruct((S, 8), _F32),
            jax.ShapeDtypeStruct((S, 8), _F32),
        ],
    )(xf, attn, wo, ln2, s_gate, s_up, s_down, rw, rb)


BT = 256
NB = (S * 2) // BT + NR
NSLOT = NB * BT
SC_NC = 2
SC_NS = 16
SC_NW = SC_NC * SC_NS
GPW = NSLOT // SC_NW
DP = 640
TPW = S // SC_NW


def _sc_mesh():
    return plsc.VectorSubcoreMesh(core_axis_name="c", subcore_axis_name="s",
                                  num_cores=SC_NC, num_subcores=SC_NS)


def _route_body(ind_ref, w_ref, sa_ref, sb_ref, wa_ref, wb_ref, be_ref):
    ind = ind_ref[...]
    indb = ind.astype(jnp.bfloat16)
    ri = jax.lax.broadcasted_iota(jnp.int32, (S, S), 0)
    ci = jax.lax.broadcasted_iota(jnp.int32, (S, S), 1)
    ltri = (ri > ci).astype(jnp.bfloat16)
    pfx = jnp.dot(ltri, indb, preferred_element_type=_F32)
    cnt = jnp.sum(ind, axis=0, keepdims=True)
    nb = jnp.floor((cnt + (BT - 1)) * (1.0 / BT))
    e1 = jax.lax.broadcasted_iota(jnp.int32, (8, 8), 0)
    e2 = jax.lax.broadcasted_iota(jnp.int32, (8, 8), 1)
    m8 = (e1 < e2).astype(_F32)
    base = jnp.dot(nb, m8, preferred_element_type=_F32) * BT
    slot = base + pfx
    colid = jax.lax.broadcasted_iota(jnp.int32, (S, 8), 1)
    sel = ind > 0.5
    cf = jnp.min(jnp.where(sel, colid, 127), axis=1, keepdims=True)
    cs = jnp.max(jnp.where(sel, colid, -1), axis=1, keepdims=True)
    w = w_ref[...]
    sa_ref[...] = jnp.sum(jnp.where(colid == cf, slot, 0.0), axis=1,
                          keepdims=True).astype(jnp.int32)
    sb_ref[...] = jnp.sum(jnp.where(colid == cs, slot, 0.0), axis=1,
                          keepdims=True).astype(jnp.int32)
    wa_ref[...] = jnp.sum(jnp.where(colid == cf, w, 0.0), axis=1, keepdims=True)
    wb_ref[...] = jnp.sum(jnp.where(colid == cs, w, 0.0), axis=1, keepdims=True)
    bid = jax.lax.broadcasted_iota(jnp.int32, (NB, 8), 0).astype(_F32) * BT
    ecol = jax.lax.broadcasted_iota(jnp.int32, (NB, 8), 1)
    inblk = (bid >= base) & (bid < base + nb * BT)
    be_ref[...] = jnp.sum(jnp.where(inblk, ecol, 0), axis=1,
                          keepdims=True).astype(jnp.int32)


def _route(ind, w):
    return pl.pallas_call(
        _route_body,
        grid=(1,),
        in_specs=[pl.BlockSpec((S, 8), lambda i: (0, 0)),
                  pl.BlockSpec((S, 8), lambda i: (0, 0))],
        out_specs=[pl.BlockSpec((S, 1), lambda i: (0, 0)),
                   pl.BlockSpec((S, 1), lambda i: (0, 0)),
                   pl.BlockSpec((S, 1), lambda i: (0, 0)),
                   pl.BlockSpec((S, 1), lambda i: (0, 0)),
                   pl.BlockSpec((NB, 1), lambda i: (0, 0))],
        out_shape=[
            jax.ShapeDtypeStruct((S, 1), jnp.int32),
            jax.ShapeDtypeStruct((S, 1), jnp.int32),
            jax.ShapeDtypeStruct((S, 1), _F32),
            jax.ShapeDtypeStruct((S, 1), _F32),
            jax.ShapeDtypeStruct((NB, 1), jnp.int32),
        ],
    )(ind, w)


def _invert_body(sa_ref, sb_ref, wat_ref, wbt_ref, tok_ref, ws_ref):
    b = pl.program_id(0)
    scol = b * 128 + jax.lax.broadcasted_iota(jnp.int32, (1, 128), 1)
    mask_a = (sa_ref[...] == scol).astype(_F32)
    mask_b = (sb_ref[...] == scol).astype(_F32)
    trow = jax.lax.broadcasted_iota(jnp.int32, (1, S), 1).astype(_F32)
    hp = jax.lax.Precision.HIGHEST
    tok = (jnp.dot(trow, mask_a, preferred_element_type=_F32, precision=hp)
           + jnp.dot(trow, mask_b, preferred_element_type=_F32, precision=hp))
    ws = (jnp.dot(wat_ref[...], mask_a, preferred_element_type=_F32,
                  precision=hp)
          + jnp.dot(wbt_ref[...], mask_b, preferred_element_type=_F32,
                    precision=hp))
    tok_ref[...] = tok.astype(jnp.int32)
    ws_ref[...] = ws


def _invert(sa, sb, wa, wb):
    col_spec = pl.BlockSpec((S, 1), lambda b: (0, 0))
    row_spec = pl.BlockSpec((1, S), lambda b: (0, 0))
    out_spec = pl.BlockSpec((1, 128), lambda b: (0, b))
    return pl.pallas_call(
        _invert_body,
        grid=(NSLOT // 128,),
        in_specs=[col_spec, col_spec, row_spec, row_spec],
        out_specs=[out_spec, out_spec],
        out_shape=[jax.ShapeDtypeStruct((1, NSLOT), jnp.int32),
                   jax.ShapeDtypeStruct((1, NSLOT), _F32)],
    )(sa, sb, wa.reshape(1, S), wb.reshape(1, S))


def _gatherx_body(h2_hbm, tok_hbm, xg_hbm, idx_v, rows_v, sem):
    wid = jax.lax.axis_index("s") * SC_NC + jax.lax.axis_index("c")
    base = wid * GPW
    pltpu.sync_copy(tok_hbm.at[pl.ds(base, GPW)], idx_v)
    pltpu.async_copy(h2_hbm.at[idx_v], rows_v, sem).wait()
    pltpu.sync_copy(rows_v, xg_hbm.at[pl.ds(base, GPW)])


def _gatherx(h2, tok):
    return pl.kernel(
        _gatherx_body,
        out_type=jax.ShapeDtypeStruct((NSLOT, DP), _F32),
        mesh=_sc_mesh(),
        scratch_types=[pltpu.VMEM((GPW,), jnp.int32),
                       pltpu.VMEM((GPW, DP), _F32),
                       pltpu.SemaphoreType.DMA],
    )(h2, tok)


def _gathery_body(yg_hbm, sa_hbm, sb_hbm, y1_hbm, y2_hbm,
                  ia_v, ib_v, ra_v, rb_v, sem):
    wid = jax.lax.axis_index("s") * SC_NC + jax.lax.axis_index("c")
    base = wid * TPW
    pltpu.sync_copy(sa_hbm.at[pl.ds(base, TPW)], ia_v)
    pltpu.sync_copy(sb_hbm.at[pl.ds(base, TPW)], ib_v)
    d1 = pltpu.async_copy(yg_hbm.at[ia_v], ra_v, sem)
    d2 = pltpu.async_copy(yg_hbm.at[ib_v], rb_v, sem)
    d1.wait()
    d2.wait()
    pltpu.sync_copy(ra_v, y1_hbm.at[pl.ds(base, TPW)])
    pltpu.sync_copy(rb_v, y2_hbm.at[pl.ds(base, TPW)])


def _gathery(yg, sa, sb):
    return pl.kernel(
        _gathery_body,
        out_type=[jax.ShapeDtypeStruct((S, DP), _F32),
                  jax.ShapeDtypeStruct((S, DP), _F32)],
        mesh=_sc_mesh(),
        scratch_types=[pltpu.VMEM((TPW,), jnp.int32),
                       pltpu.VMEM((TPW,), jnp.int32),
                       pltpu.VMEM((TPW, DP), _F32),
                       pltpu.VMEM((TPW, DP), _F32),
                       pltpu.SemaphoreType.DMA],
    )(yg, sa, sb)


def _group_body(be_ref, xg_ref, ws_ref, rg_ref, ru_ref, rd_ref, yg_ref):
    xgb = xg_ref[:, :D]
    g = jnp.dot(xgb, rg_ref[0], preferred_element_type=_F32)
    u = jnp.dot(xgb, ru_ref[0], preferred_element_type=_F32)
    a = g * jax.nn.sigmoid(g) * u
    res = jnp.dot(a, rd_ref[0], preferred_element_type=_F32) * ws_ref[...]
    yg_ref[...] = jnp.concatenate([res, jnp.zeros((BT, DP - D), _F32)], axis=1)


def _group(be, xg, ws, r_gate, r_up, r_down):
    grid_spec = pltpu.PrefetchScalarGridSpec(
        num_scalar_prefetch=1,
        grid=(NB,),
        in_specs=[
            pl.BlockSpec((BT, DP), lambda b, be: (b, 0)),
            pl.BlockSpec((BT, 1), lambda b, be: (b, 0)),
            pl.BlockSpec((1, D, INTER), lambda b, be: (be[b], 0, 0)),
            pl.BlockSpec((1, D, INTER), lambda b, be: (be[b], 0, 0)),
            pl.BlockSpec((1, INTER, D), lambda b, be: (be[b], 0, 0)),
        ],
        out_specs=pl.BlockSpec((BT, DP), lambda b, be: (b, 0)),
    )
    return pl.pallas_call(
        _group_body,
        grid_spec=grid_spec,
        out_shape=jax.ShapeDtypeStruct((NSLOT, DP), _F32),
    )(be, xg, ws, r_gate, r_up, r_down)


def _combine_body(part_ref, y1_ref, y2_ref, out_ref):
    out_ref[...] = part_ref[...] + y1_ref[:, :D] + y2_ref[:, :D]


def _combine(part, y1, y2):
    row_spec = pl.BlockSpec((BC, D), lambda i: (i, 0))
    pad_spec = pl.BlockSpec((BC, DP), lambda i: (i, 0))
    return pl.pallas_call(
        _combine_body,
        grid=(S // BC,),
        in_specs=[row_spec, pad_spec, pad_spec],
        out_specs=row_spec,
        out_shape=jax.ShapeDtypeStruct((S, D), _F32),
    )(part, y1, y2)


def kernel(x, ln1_w, ln2_w, wq_d, wkv_d, wq_u, wk_u, wv_u, wo, s_gate, s_up,
           s_down, r_gate, r_up, r_down, router_w, routing_bias):
    xf = x.reshape(S, D)
    ln1 = ln1_w.reshape(1, D)
    ln2 = ln2_w.reshape(1, D)
    rw = jnp.pad(router_w, ((0, 0), (0, 1)))
    rb = jnp.pad(routing_bias, (0, 1)).reshape(1, 8)

    q, k, v = _qkv(xf, ln1, wq_d, wq_u, wkv_d, wk_u, wv_u)
    attn = _attention(q, k, v)
    part, h2, w, ind = _post(xf, attn, wo, ln2, s_gate, s_up, s_down, rw, rb)
    sa, sb, wa, wb, be = _route(ind, w)
    sa1 = sa.reshape(S)
    sb1 = sb.reshape(S)
    tok, ws = _invert(sa, sb, wa, wb)
    xg = _gatherx(h2, tok.reshape(NSLOT))
    yg = _group(be.reshape(NB), xg, ws.reshape(NSLOT, 1), r_gate, r_up, r_down)
    y1, y2 = _gathery(yg, sa1, sb1)
    out = _combine(part, y1, y2)
    return out.reshape(1, S, D)


# speedup vs baseline: 1.6103x; 1.6103x over previous
import jax
import jax.numpy as jnp
from jax.experimental import pallas as pl

S = 2048
D = 576
H = 9
HD = D // H
LAT = D // 4
INTER = 1536
NR = 7
EPS = 1e-5

BA = 512
BQ = 512
BK = 512
BC = 512
FB = 512
NF = INTER // FB

_F32 = jnp.float32


def _rope(t, cos, sin):
    parts = []
    for h in range(H):
        th = t[:, h * HD:(h + 1) * HD]
        rot = jnp.concatenate([-th[:, HD // 2:], th[:, :HD // 2]], axis=1)
        parts.append(th * cos + rot * sin)
    return jnp.concatenate(parts, axis=1)


def _qkv_body(x_ref, ln1_ref, wqd_ref, wqu_ref, wkvd_ref, wku_ref, wvu_ref,
              q_ref, k_ref, v_ref):
    i = pl.program_id(0)
    xb = x_ref[...]
    h = xb * jax.lax.rsqrt(jnp.mean(xb * xb, axis=1, keepdims=True) + EPS)
    h = h * ln1_ref[...]
    q = jnp.dot(jnp.dot(h, wqd_ref[...], preferred_element_type=_F32),
                wqu_ref[...], preferred_element_type=_F32)
    kv = jnp.dot(h, wkvd_ref[...], preferred_element_type=_F32)
    k = jnp.dot(kv, wku_ref[...], preferred_element_type=_F32)
    v = jnp.dot(kv, wvu_ref[...], preferred_element_type=_F32)
    pos = (i * BA + jax.lax.broadcasted_iota(jnp.int32, (BA, HD), 0)).astype(_F32)
    lane = jax.lax.broadcasted_iota(jnp.int32, (BA, HD), 1)
    kk = jnp.where(lane < HD // 2, lane, lane - HD // 2).astype(_F32)
    inv = jnp.exp(kk * (-2.0 * jnp.log(10000.0) / HD))
    ang = pos * inv
    cos = jnp.cos(ang)
    sin = jnp.sin(ang)
    q_ref[...] = _rope(q, cos, sin).astype(jnp.bfloat16)
    k_ref[...] = _rope(k, cos, sin).astype(jnp.bfloat16)
    v_ref[...] = v.astype(jnp.bfloat16)


def _qkv(xf, ln1, wq_d, wq_u, wkv_d, wk_u, wv_u):
    out = jax.ShapeDtypeStruct((S, D), jnp.bfloat16)
    row_spec = pl.BlockSpec((BA, D), lambda i: (i, 0))
    return pl.pallas_call(
        _qkv_body,
        grid=(S // BA,),
        in_specs=[
            row_spec,
            pl.BlockSpec((1, D), lambda i: (0, 0)),
            pl.BlockSpec((D, LAT), lambda i: (0, 0)),
            pl.BlockSpec((LAT, D), lambda i: (0, 0)),
            pl.BlockSpec((D, LAT), lambda i: (0, 0)),
            pl.BlockSpec((LAT, D), lambda i: (0, 0)),
            pl.BlockSpec((LAT, D), lambda i: (0, 0)),
        ],
        out_specs=[row_spec, row_spec, row_spec],
        out_shape=[out, out, out],
    )(xf, ln1, wq_d, wq_u, wkv_d, wk_u, wv_u)


def _attn_body(q_ref, k_ref, v_ref, o_ref):
    i = pl.program_id(0)
    rows = jax.lax.broadcasted_iota(jnp.int32, (BQ, BK), 0)
    cols = jax.lax.broadcasted_iota(jnp.int32, (BQ, BK), 1)
    for h in range(H):
        hs = slice(h * HD, (h + 1) * HD)
        qh = q_ref[:, hs]

        def body(j, carry, qh=qh, hs=hs):
            m, l, acc = carry
            kh = k_ref[pl.ds(j * BK, BK), hs]
            lg = jax.lax.dot_general(qh, kh, (((1,), (1,)), ((), ())),
                                     preferred_element_type=_F32) * 0.125
            mask = (i * BQ + rows) >= (j * BK + cols)
            lg = jnp.where(mask, lg, -1e30)
            mj = jnp.max(lg, axis=1, keepdims=True)
            mn = jnp.maximum(m, mj)
            p = jnp.exp(lg - mn)
            scale = jnp.exp(m - mn)
            l2 = l * scale + jnp.sum(p, axis=1, keepdims=True)
            acc2 = acc * scale + jnp.dot(
                p.astype(jnp.bfloat16), v_ref[pl.ds(j * BK, BK), hs],
                preferred_element_type=_F32)
            return mn, l2, acc2

        m0 = jnp.full((BQ, 1), -1e30, _F32)
        l0 = jnp.zeros((BQ, 1), _F32)
        a0 = jnp.zeros((BQ, HD), _F32)
        m, l, acc = jax.lax.fori_loop(0, i + 1, body, (m0, l0, a0))
        o_ref[:, hs] = (acc / l).astype(jnp.bfloat16)


def _attention(q, k, v):
    row_spec = pl.BlockSpec((BQ, D), lambda i: (i, 0))
    kv_spec = pl.BlockSpec((S, D), lambda i: (0, 0))
    return pl.pallas_call(
        _attn_body,
        grid=(S // BQ,),
        in_specs=[row_spec, kv_spec, kv_spec],
        out_specs=row_spec,
        out_shape=jax.ShapeDtypeStruct((S, D), jnp.bfloat16),
    )(q, k, v)


def _post_body(x_ref, attn_ref, wo_ref, ln2_ref, sg_ref, su_ref, sd_ref,
               rw_ref, rb_ref, part_ref, h2_ref, w_ref):
    x2 = x_ref[...] + jnp.dot(attn_ref[...].astype(_F32), wo_ref[...],
                              preferred_element_type=_F32)
    h2 = x2 * jax.lax.rsqrt(jnp.mean(x2 * x2, axis=1, keepdims=True) + EPS)
    h2 = h2 * ln2_ref[...]
    g = jnp.dot(h2, sg_ref[...], preferred_element_type=_F32)
    u = jnp.dot(h2, su_ref[...], preferred_element_type=_F32)
    a = g * jax.nn.sigmoid(g) * u
    shared = jnp.dot(a, sd_ref[...], preferred_element_type=_F32)
    part_ref[...] = x2 + shared
    h2_ref[...] = h2.astype(jnp.bfloat16)
    logits = jnp.dot(h2, rw_ref[...], preferred_element_type=_F32) + rb_ref[...]
    p = jax.nn.sigmoid(logits)
    colid = jax.lax.broadcasted_iota(jnp.int32, (BC, 8), 1)
    p = jnp.where(colid < NR, p, -1.0)
    m1 = jnp.max(p, axis=1, keepdims=True)
    i1 = jnp.min(jnp.where(p == m1, colid, 127), axis=1, keepdims=True)
    pm = jnp.where(colid == i1, -1.0, p)
    m2 = jnp.max(pm, axis=1, keepdims=True)
    i2 = jnp.min(jnp.where(pm == m2, colid, 127), axis=1, keepdims=True)
    den = m1 + m2
    w_ref[...] = (jnp.where(colid == i1, m1, 0.0)
                  + jnp.where(colid == i2, m2, 0.0)) / den


def _post(xf, attn, wo, ln2, s_gate, s_up, s_down, rw, rb):
    row_spec = pl.BlockSpec((BC, D), lambda i: (i, 0))
    return pl.pallas_call(
        _post_body,
        grid=(S // BC,),
        in_specs=[
            row_spec,
            row_spec,
            pl.BlockSpec((D, D), lambda i: (0, 0)),
            pl.BlockSpec((1, D), lambda i: (0, 0)),
            pl.BlockSpec((D, INTER), lambda i: (0, 0)),
            pl.BlockSpec((D, INTER), lambda i: (0, 0)),
            pl.BlockSpec((INTER, D), lambda i: (0, 0)),
            pl.BlockSpec((D, 8), lambda i: (0, 0)),
            pl.BlockSpec((1, 8), lambda i: (0, 0)),
        ],
        out_specs=[row_spec, row_spec, pl.BlockSpec((BC, 8), lambda i: (i, 0))],
        out_shape=[
            jax.ShapeDtypeStruct((S, D), _F32),
            jax.ShapeDtypeStruct((S, D), jnp.bfloat16),
            jax.ShapeDtypeStruct((S, 8), _F32),
        ],
    )(xf, attn, wo, ln2, s_gate, s_up, s_down, rw, rb)


def _moe_body(h2_ref, w_ref, part_ref, rg_ref, ru_ref, rd_ref, out_ref):
    e = pl.program_id(0)
    f = pl.program_id(1)

    @pl.when((e == 0) & (f == 0))
    def _init():
        out_ref[...] = part_ref[...]

    h2 = h2_ref[...]
    g = jnp.dot(h2.astype(_F32), rg_ref[0], preferred_element_type=_F32)
    u = jnp.dot(h2.astype(_F32), ru_ref[0], preferred_element_type=_F32)
    a = g * jax.nn.sigmoid(g) * u
    pp = jnp.dot(a, rd_ref[0], preferred_element_type=_F32)
    wf = w_ref[...]
    we = jnp.zeros((S, 1), _F32)
    for j in range(NR):
        we = jnp.where(e == j, wf[:, j:j + 1], we)
    out_ref[...] += pp * we


def _moe(h2, w, part, r_gate, r_up, r_down):
    full_spec = pl.BlockSpec((S, D), lambda e, f: (0, 0))
    return pl.pallas_call(
        _moe_body,
        grid=(NR, NF),
        in_specs=[
            full_spec,
            pl.BlockSpec((S, 8), lambda e, f: (0, 0)),
            full_spec,
            pl.BlockSpec((1, D, FB), lambda e, f: (e, 0, f)),
            pl.BlockSpec((1, D, FB), lambda e, f: (e, 0, f)),
            pl.BlockSpec((1, FB, D), lambda e, f: (e, f, 0)),
        ],
        out_specs=full_spec,
        out_shape=jax.ShapeDtypeStruct((S, D), _F32),
    )(h2, w, part, r_gate, r_up, r_down)


def kernel(x, ln1_w, ln2_w, wq_d, wkv_d, wq_u, wk_u, wv_u, wo, s_gate, s_up,
           s_down, r_gate, r_up, r_down, router_w, routing_bias):
    xf = x.reshape(S, D)
    ln1 = ln1_w.reshape(1, D)
    ln2 = ln2_w.reshape(1, D)
    rw = jnp.pad(router_w, ((0, 0), (0, 1)))
    rb = jnp.pad(routing_bias, (0, 1)).reshape(1, 8)

    q, k, v = _qkv(xf, ln1, wq_d, wq_u, wkv_d, wk_u, wv_u)
    attn = _attention(q, k, v)
    part, h2, w = _post(xf, attn, wo, ln2, s_gate, s_up, s_down, rw, rb)
    out = _moe(h2, w, part, r_gate, r_up, r_down)
    return out.reshape(1, S, D)
